# baseline (speedup 1.0000x reference)
import functools

import jax
import jax.numpy as jnp
from jax import lax
from jax.experimental import pallas as pl
from jax.experimental.pallas import tpu as pltpu
from jax.experimental.pallas import tpu_sc as plsc

N = 10000
E = 320000
F = 128

NC = 2
NS = 16
NW = NC * NS
EPT = E // NW
C = 80
NCHUNK = EPT // C
NPAD = 10240
RPT = NPAD // NS
ZROWS = 32
NBUF = 4
NIB = 5
CPT = NPAD // NS


def _make_sc_agg(with_counts: bool):
    mesh = plsc.VectorSubcoreMesh(core_axis_name="c", subcore_axis_name="s")
    out_type = [jax.ShapeDtypeStruct((NC, NPAD, F), jnp.float32)]
    if with_counts:
        out_type.append(jax.ShapeDtypeStruct((NC, NPAD), jnp.float32))

    scratch = [
        pltpu.VMEM_SHARED((NPAD, F), jnp.float32),
        pltpu.VMEM_SHARED((NPAD,), jnp.float32),
        pltpu.VMEM((ZROWS, F), jnp.float32),
        pltpu.VMEM((CPT,), jnp.float32),
        pltpu.VMEM((NIB, C), jnp.int32),
        pltpu.VMEM((NIB, C), jnp.int32),
        pltpu.VMEM((NBUF, C, F), jnp.float32),
        pltpu.VMEM((C,), jnp.float32),
        pltpu.SemaphoreType.DMA,
        pltpu.SemaphoreType.DMA,
        pltpu.SemaphoreType.DMA,
        pltpu.SemaphoreType.DMA,
        pltpu.SemaphoreType.DMA,
    ]

    @functools.partial(
        pl.kernel, mesh=mesh, out_type=tuple(out_type),
        scratch_types=tuple(scratch),
    )
    def sc_agg(x_hbm, src_hbm, dst_hbm, *rest):
        if with_counts:
            sums_hbm, cnt_hbm = rest[0], rest[1]
            rest = rest[2:]
        else:
            sums_hbm, cnt_hbm = rest[0], None
            rest = rest[1:]
        (acc_sh, cnt_sh, zb, zc, sbuf, dbuf, rows, ones_v,
         isem, gsem, zsem, ssem, csem) = rest

        cid = lax.axis_index("c")
        sid = lax.axis_index("s")
        wid = sid * NC + cid

        ebase = wid * EPT

        def fire_idx(g):
            base = ebase + g * C
            b = g % NIB
            pltpu.async_copy(src_hbm.at[pl.ds(base, C)], sbuf.at[b], isem)
            pltpu.async_copy(dst_hbm.at[pl.ds(base, C)], dbuf.at[b], isem)

        def wait_idx():
            pltpu.make_async_copy(src_hbm.at[pl.ds(0, C)], sbuf.at[0],
                                  isem).wait()
            pltpu.make_async_copy(dst_hbm.at[pl.ds(0, C)], dbuf.at[0],
                                  isem).wait()

        fire_idx(0)
        fire_idx(1)
        fire_idx(2)

        z16 = jnp.zeros((16,), jnp.float32)

        def zb_body(i, _):
            zb[i // (F // 16), pl.ds((i % (F // 16)) * 16, 16)] = z16
            return 0
        lax.fori_loop(0, ZROWS * (F // 16), zb_body, 0)

        if with_counts:
            def zc_body(i, _):
                zc[pl.ds(i * 16, 16)] = z16
                return 0
            lax.fori_loop(0, CPT // 16, zc_body, 0)

            def ones_body(i, _):
                ones_v[pl.ds(i * 16, 16)] = jnp.ones((16,), jnp.float32)
                return 0
            lax.fori_loop(0, C // 16, ones_body, 0)

        r0 = sid * RPT
        for r in range(RPT // ZROWS):
            pltpu.async_copy(zb, acc_sh.at[pl.ds(r0 + r * ZROWS, ZROWS)],
                             zsem)
        c0 = sid * CPT
        if with_counts:
            pltpu.async_copy(zc, cnt_sh.at[pl.ds(c0, CPT)], zsem)
        for r in range(RPT // ZROWS):
            pltpu.make_async_copy(zb, acc_sh.at[pl.ds(r0, ZROWS)],
                                  zsem).wait()
        if with_counts:
            pltpu.make_async_copy(zc, cnt_sh.at[pl.ds(c0, CPT)], zsem).wait()

        plsc.subcore_barrier()

        def fire_gather(g):
            pltpu.async_copy(x_hbm.at[sbuf.at[g % NIB]], rows.at[g % NBUF],
                             gsem)

        def wait_gather(g):
            pltpu.make_async_copy(x_hbm.at[sbuf.at[g % NIB]],
                                  rows.at[g % NBUF], gsem).wait()

        def fire_scat(g):
            pltpu.async_copy(rows.at[g % NBUF], acc_sh.at[dbuf.at[g % NIB]],
                             ssem, add=True)
            if with_counts:
                pltpu.async_copy(ones_v, cnt_sh.at[dbuf.at[g % NIB]], csem,
                                 add=True)

        def wait_scat(g):
            pltpu.make_async_copy(rows.at[g % NBUF],
                                  acc_sh.at[dbuf.at[g % NIB]], ssem).wait()
            if with_counts:
                pltpu.make_async_copy(ones_v, cnt_sh.at[dbuf.at[g % NIB]],
                                      csem).wait()

        wait_idx()
        fire_gather(0)
        wait_idx()
        fire_gather(1)

        def chunk_body(g, _):
            wait_gather(g)

            @pl.when(g + 3 < NCHUNK)
            def _():
                fire_idx(g + 3)

            @pl.when(g >= 2)
            def _():
                wait_scat(g - 2)

            @pl.when(g + 2 < NCHUNK)
            def _():
                wait_idx()
                fire_gather(g + 2)

            fire_scat(g)
            return 0
        lax.fori_loop(0, NCHUNK, chunk_body, 0)
        wait_scat(NCHUNK - 2)
        wait_scat(NCHUNK - 1)

        plsc.subcore_barrier()

        pltpu.sync_copy(acc_sh.at[pl.ds(r0, RPT)],
                        sums_hbm.at[cid, pl.ds(r0, RPT)])
        if with_counts:
            pltpu.sync_copy(cnt_sh.at[pl.ds(c0, CPT)],
                            cnt_hbm.at[cid, pl.ds(c0, CPT)])

    return sc_agg


_sc_agg_counts = _make_sc_agg(True)
_sc_agg_nocounts = _make_sc_agg(False)


def _make_tc_mm(relu: bool):
    R = 1000
    grid = (N // R,)

    def mm_body(s0_ref, s1_ref, x_ref, c0_ref, c1_ref, wl_ref, wr_ref, b_ref,
                o_ref):
        c = c0_ref[...] + c1_ref[...]
        scale = 1.0 / jnp.maximum(c, 1.0)
        agg = (s0_ref[...] + s1_ref[...]) * scale
        out = (jnp.dot(agg, wl_ref[...], preferred_element_type=jnp.float32)
               + jnp.dot(x_ref[...], wr_ref[...],
                         preferred_element_type=jnp.float32)
               + b_ref[...])
        if relu:
            out = jnp.maximum(out, 0.0)
        o_ref[...] = out

    row_spec = pl.BlockSpec((R, F), lambda i: (i, 0))
    col_spec = pl.BlockSpec((R, 1), lambda i: (i, 0))
    full_spec = pl.BlockSpec((F, F), lambda i: (0, 0))
    bias_spec = pl.BlockSpec((1, F), lambda i: (0, 0))

    return pl.pallas_call(
        mm_body,
        grid=grid,
        in_specs=[row_spec, row_spec, row_spec, col_spec, col_spec,
                  full_spec, full_spec, bias_spec],
        out_specs=row_spec,
        out_shape=jax.ShapeDtypeStruct((N, F), jnp.float32),
    )


_tc_mm_relu = _make_tc_mm(True)
_tc_mm = _make_tc_mm(False)


def kernel(x, edge_index, W1l, W1r, b1, W2l, W2r, b2):
    src = edge_index[0].astype(jnp.int32)
    dst = edge_index[1].astype(jnp.int32)
    b1r = b1.reshape(1, F)
    b2r = b2.reshape(1, F)

    sums1, cnt = _sc_agg_counts(x, src, dst)
    c0 = cnt[0, :N].reshape(N, 1)
    c1 = cnt[1, :N].reshape(N, 1)
    h = _tc_mm_relu(sums1[0, :N], sums1[1, :N], x, c0, c1, W1l, W1r, b1r)

    (sums2,) = _sc_agg_nocounts(h, src, dst)
    out = _tc_mm(sums2[0, :N], sums2[1, :N], h, c0, c1, W2l, W2r, b2r)
    return out


# speedup vs baseline: 1.0218x; 1.0218x over previous
import functools

import jax
import jax.numpy as jnp
from jax import lax
from jax.experimental import pallas as pl
from jax.experimental.pallas import tpu as pltpu
from jax.experimental.pallas import tpu_sc as plsc

N = 10000
E = 320000
F = 128

NC = 2
NS = 16
NW = NC * NS
EPT = E // NW
C = 80
NCHUNK = EPT // C
NPAD = 10240
RPT = NPAD // NS
ZROWS = 32
NBUF = 4
NIB = 5
CPT = NPAD // NS


def _make_sc_agg(with_counts: bool):
    mesh = plsc.VectorSubcoreMesh(core_axis_name="c", subcore_axis_name="s")
    out_type = [jax.ShapeDtypeStruct((NC, NPAD, F), jnp.float32)]
    if with_counts:
        out_type.append(jax.ShapeDtypeStruct((NC, NPAD), jnp.float32))

    scratch = [
        pltpu.VMEM_SHARED((NPAD, F), jnp.float32),
        pltpu.VMEM_SHARED((NPAD,), jnp.float32),
        pltpu.VMEM((ZROWS, F), jnp.float32),
        pltpu.VMEM((CPT,), jnp.float32),
        pltpu.VMEM((NIB, C), jnp.int32),
        pltpu.VMEM((NIB, C), jnp.int32),
        pltpu.VMEM((NBUF, C, F), jnp.float32),
        pltpu.VMEM((C,), jnp.float32),
        pltpu.SemaphoreType.DMA,
        pltpu.SemaphoreType.DMA,
        pltpu.SemaphoreType.DMA,
        pltpu.SemaphoreType.DMA,
        pltpu.SemaphoreType.DMA,
    ]

    @functools.partial(
        pl.kernel, mesh=mesh, out_type=tuple(out_type),
        scratch_types=tuple(scratch),
    )
    def sc_agg(x_hbm, src_hbm, dst_hbm, *rest):
        if with_counts:
            sums_hbm, cnt_hbm = rest[0], rest[1]
            rest = rest[2:]
        else:
            sums_hbm, cnt_hbm = rest[0], None
            rest = rest[1:]
        (acc_sh, cnt_sh, zb, zc, sbuf, dbuf, rows, ones_v,
         isem, gsem, zsem, ssem, csem) = rest

        cid = lax.axis_index("c")
        sid = lax.axis_index("s")
        wid = sid * NC + cid

        ebase = wid * EPT

        def fire_idx(g):
            base = ebase + g * C
            b = g % NIB
            pltpu.async_copy(src_hbm.at[pl.ds(base, C)], sbuf.at[b], isem)
            pltpu.async_copy(dst_hbm.at[pl.ds(base, C)], dbuf.at[b], isem)

        def wait_idx():
            pltpu.make_async_copy(src_hbm.at[pl.ds(0, C)], sbuf.at[0],
                                  isem).wait()
            pltpu.make_async_copy(dst_hbm.at[pl.ds(0, C)], dbuf.at[0],
                                  isem).wait()

        fire_idx(0)
        fire_idx(1)
        fire_idx(2)

        z16 = jnp.zeros((16,), jnp.float32)

        def zb_body(i, _):
            zb[i // (F // 16), pl.ds((i % (F // 16)) * 16, 16)] = z16
            return 0
        lax.fori_loop(0, ZROWS * (F // 16), zb_body, 0)

        if with_counts:
            def zc_body(i, _):
                zc[pl.ds(i * 16, 16)] = z16
                return 0
            lax.fori_loop(0, CPT // 16, zc_body, 0)

            def ones_body(i, _):
                ones_v[pl.ds(i * 16, 16)] = jnp.ones((16,), jnp.float32)
                return 0
            lax.fori_loop(0, C // 16, ones_body, 0)

        r0 = sid * RPT
        for r in range(RPT // ZROWS):
            pltpu.async_copy(zb, acc_sh.at[pl.ds(r0 + r * ZROWS, ZROWS)],
                             zsem)
        c0 = sid * CPT
        if with_counts:
            pltpu.async_copy(zc, cnt_sh.at[pl.ds(c0, CPT)], zsem)
        for r in range(RPT // ZROWS):
            pltpu.make_async_copy(zb, acc_sh.at[pl.ds(r0, ZROWS)],
                                  zsem).wait()
        if with_counts:
            pltpu.make_async_copy(zc, cnt_sh.at[pl.ds(c0, CPT)], zsem).wait()

        plsc.subcore_barrier()

        def fire_gather(g):
            pltpu.async_copy(x_hbm.at[sbuf.at[g % NIB]], rows.at[g % NBUF],
                             gsem)

        def wait_gather(g):
            pltpu.make_async_copy(x_hbm.at[sbuf.at[g % NIB]],
                                  rows.at[g % NBUF], gsem).wait()

        def scat(g):
            pltpu.sync_copy(rows.at[g % NBUF], acc_sh.at[dbuf.at[g % NIB]],
                            add=True)
            if with_counts:
                pltpu.sync_copy(ones_v, cnt_sh.at[dbuf.at[g % NIB]],
                                add=True)

        wait_idx()
        fire_gather(0)
        wait_idx()
        fire_gather(1)

        def chunk_body(g, _):
            wait_gather(g)

            @pl.when(g + 3 < NCHUNK)
            def _():
                fire_idx(g + 3)

            @pl.when(g + 2 < NCHUNK)
            def _():
                wait_idx()
                fire_gather(g + 2)

            scat(g)
            return 0
        lax.fori_loop(0, NCHUNK, chunk_body, 0)

        plsc.subcore_barrier()

        pltpu.sync_copy(acc_sh.at[pl.ds(r0, RPT)],
                        sums_hbm.at[cid, pl.ds(r0, RPT)])
        if with_counts:
            pltpu.sync_copy(cnt_sh.at[pl.ds(c0, CPT)],
                            cnt_hbm.at[cid, pl.ds(c0, CPT)])

    return sc_agg


_sc_agg_counts = _make_sc_agg(True)
_sc_agg_nocounts = _make_sc_agg(False)


def _make_tc_mm(relu: bool):
    R = 1000
    grid = (N // R,)

    def mm_body(s0_ref, s1_ref, x_ref, c0_ref, c1_ref, wl_ref, wr_ref, b_ref,
                o_ref):
        c = c0_ref[...] + c1_ref[...]
        scale = 1.0 / jnp.maximum(c, 1.0)
        agg = (s0_ref[...] + s1_ref[...]) * scale
        out = (jnp.dot(agg, wl_ref[...], preferred_element_type=jnp.float32)
               + jnp.dot(x_ref[...], wr_ref[...],
                         preferred_element_type=jnp.float32)
               + b_ref[...])
        if relu:
            out = jnp.maximum(out, 0.0)
        o_ref[...] = out

    row_spec = pl.BlockSpec((R, F), lambda i: (i, 0))
    col_spec = pl.BlockSpec((R, 1), lambda i: (i, 0))
    full_spec = pl.BlockSpec((F, F), lambda i: (0, 0))
    bias_spec = pl.BlockSpec((1, F), lambda i: (0, 0))

    return pl.pallas_call(
        mm_body,
        grid=grid,
        in_specs=[row_spec, row_spec, row_spec, col_spec, col_spec,
                  full_spec, full_spec, bias_spec],
        out_specs=row_spec,
        out_shape=jax.ShapeDtypeStruct((N, F), jnp.float32),
    )


_tc_mm_relu = _make_tc_mm(True)
_tc_mm = _make_tc_mm(False)


def kernel(x, edge_index, W1l, W1r, b1, W2l, W2r, b2):
    src = edge_index[0].astype(jnp.int32)
    dst = edge_index[1].astype(jnp.int32)
    b1r = b1.reshape(1, F)
    b2r = b2.reshape(1, F)

    sums1, cnt = _sc_agg_counts(x, src, dst)
    c0 = cnt[0, :N].reshape(N, 1)
    c1 = cnt[1, :N].reshape(N, 1)
    h = _tc_mm_relu(sums1[0, :N], sums1[1, :N], x, c0, c1, W1l, W1r, b1r)

    (sums2,) = _sc_agg_nocounts(h, src, dst)
    out = _tc_mm(sums2[0, :N], sums2[1, :N], h, c0, c1, W2l, W2r, b2r)
    return out


# speedup vs baseline: 1.0741x; 1.0512x over previous
import functools

import jax
import jax.numpy as jnp
from jax import lax
from jax.experimental import pallas as pl
from jax.experimental.pallas import tpu as pltpu
from jax.experimental.pallas import tpu_sc as plsc

N = 10000
E = 320000
F = 128

NC = 2
NS = 16
NW = NC * NS
EPT = E // NW
C = 80
NCHUNK = EPT // C
NPAD = 10240
RPT = NPAD // NS
ZROWS = 32
NBUF = 4
NIB = 5
CPT = NPAD // NS


def _make_sc_agg(with_counts: bool):
    mesh = plsc.VectorSubcoreMesh(core_axis_name="c", subcore_axis_name="s")
    out_type = [jax.ShapeDtypeStruct((NC, N, F), jnp.float32)]
    if with_counts:
        out_type.append(jax.ShapeDtypeStruct((NC, NPAD), jnp.float32))

    scratch = [
        pltpu.VMEM_SHARED((NPAD, F), jnp.float32),
        pltpu.VMEM_SHARED((NPAD,), jnp.float32),
        pltpu.VMEM((ZROWS, F), jnp.float32),
        pltpu.VMEM((CPT,), jnp.float32),
        pltpu.VMEM((NIB, C), jnp.int32),
        pltpu.VMEM((NIB, C), jnp.int32),
        pltpu.VMEM((NBUF, C, F), jnp.float32),
        pltpu.VMEM((C,), jnp.float32),
        pltpu.SemaphoreType.DMA,
        pltpu.SemaphoreType.DMA,
        pltpu.SemaphoreType.DMA,
        pltpu.SemaphoreType.DMA,
        pltpu.SemaphoreType.DMA,
    ]

    @functools.partial(
        pl.kernel, mesh=mesh, out_type=tuple(out_type),
        scratch_types=tuple(scratch),
    )
    def sc_agg(x_hbm, src_hbm, dst_hbm, *rest):
        if with_counts:
            sums_hbm, cnt_hbm = rest[0], rest[1]
            rest = rest[2:]
        else:
            sums_hbm, cnt_hbm = rest[0], None
            rest = rest[1:]
        (acc_sh, cnt_sh, zb, zc, sbuf, dbuf, rows, ones_v,
         isem, gsem, zsem, ssem, csem) = rest

        cid = lax.axis_index("c")
        sid = lax.axis_index("s")
        wid = sid * NC + cid

        ebase = wid * EPT

        def fire_idx(g):
            base = ebase + g * C
            b = g % NIB
            pltpu.async_copy(src_hbm.at[pl.ds(base, C)], sbuf.at[b], isem)
            pltpu.async_copy(dst_hbm.at[pl.ds(base, C)], dbuf.at[b], isem)

        def wait_idx():
            pltpu.make_async_copy(src_hbm.at[pl.ds(0, C)], sbuf.at[0],
                                  isem).wait()
            pltpu.make_async_copy(dst_hbm.at[pl.ds(0, C)], dbuf.at[0],
                                  isem).wait()

        fire_idx(0)
        fire_idx(1)
        fire_idx(2)

        z16 = jnp.zeros((16,), jnp.float32)

        def zb_body(i, _):
            zb[i // (F // 16), pl.ds((i % (F // 16)) * 16, 16)] = z16
            return 0
        lax.fori_loop(0, ZROWS * (F // 16), zb_body, 0)

        if with_counts:
            def zc_body(i, _):
                zc[pl.ds(i * 16, 16)] = z16
                return 0
            lax.fori_loop(0, CPT // 16, zc_body, 0)

            def ones_body(i, _):
                ones_v[pl.ds(i * 16, 16)] = jnp.ones((16,), jnp.float32)
                return 0
            lax.fori_loop(0, C // 16, ones_body, 0)

        r0 = sid * RPT
        for r in range(RPT // ZROWS):
            pltpu.async_copy(zb, acc_sh.at[pl.ds(r0 + r * ZROWS, ZROWS)],
                             zsem)
        c0 = sid * CPT
        if with_counts:
            pltpu.async_copy(zc, cnt_sh.at[pl.ds(c0, CPT)], zsem)
        for r in range(RPT // ZROWS):
            pltpu.make_async_copy(zb, acc_sh.at[pl.ds(r0, ZROWS)],
                                  zsem).wait()
        if with_counts:
            pltpu.make_async_copy(zc, cnt_sh.at[pl.ds(c0, CPT)], zsem).wait()

        plsc.subcore_barrier()

        def fire_gather(g):
            pltpu.async_copy(x_hbm.at[sbuf.at[g % NIB]], rows.at[g % NBUF],
                             gsem)

        def wait_gather(g):
            pltpu.make_async_copy(x_hbm.at[sbuf.at[g % NIB]],
                                  rows.at[g % NBUF], gsem).wait()

        def scat(g):
            pltpu.sync_copy(rows.at[g % NBUF], acc_sh.at[dbuf.at[g % NIB]],
                            add=True)
            if with_counts:
                pltpu.sync_copy(ones_v, cnt_sh.at[dbuf.at[g % NIB]],
                                add=True)

        wait_idx()
        fire_gather(0)
        wait_idx()
        fire_gather(1)

        def chunk_body(g, _):
            wait_gather(g)

            @pl.when(g + 3 < NCHUNK)
            def _():
                fire_idx(g + 3)

            @pl.when(g + 2 < NCHUNK)
            def _():
                wait_idx()
                fire_gather(g + 2)

            scat(g)
            return 0
        lax.fori_loop(0, NCHUNK, chunk_body, 0)

        plsc.subcore_barrier()

        if with_counts:
            pltpu.sync_copy(cnt_sh.at[pl.ds(c0, CPT)],
                            cnt_hbm.at[cid, pl.ds(c0, CPT)])

        @pl.when(sid < NS - 1)
        def _():
            pltpu.sync_copy(acc_sh.at[pl.ds(r0, RPT)],
                            sums_hbm.at[cid, pl.ds(r0, RPT)])

        @pl.when(sid == NS - 1)
        def _():
            tail = N - (NS - 1) * RPT
            pltpu.sync_copy(acc_sh.at[pl.ds((NS - 1) * RPT, tail)],
                            sums_hbm.at[cid, pl.ds((NS - 1) * RPT, tail)])

    return sc_agg


_sc_agg_counts = _make_sc_agg(True)
_sc_agg_nocounts = _make_sc_agg(False)


def _make_tc_mm(relu: bool):
    R = 1000
    grid = (N // R,)

    def mm_body(s0_ref, s1_ref, x_ref, c0_ref, c1_ref, wl_ref, wr_ref, b_ref,
                o_ref):
        c = c0_ref[0] + c1_ref[0]
        scale = 1.0 / jnp.maximum(c, 1.0)
        agg = (s0_ref[0] + s1_ref[0]) * scale
        out = (jnp.dot(agg, wl_ref[...], preferred_element_type=jnp.float32)
               + jnp.dot(x_ref[...], wr_ref[...],
                         preferred_element_type=jnp.float32)
               + b_ref[...])
        if relu:
            out = jnp.maximum(out, 0.0)
        o_ref[...] = out

    half0_spec = pl.BlockSpec((1, R, F), lambda i: (0, i, 0))
    half1_spec = pl.BlockSpec((1, R, F), lambda i: (1, i, 0))
    col0_spec = pl.BlockSpec((1, R, 1), lambda i: (0, i, 0))
    col1_spec = pl.BlockSpec((1, R, 1), lambda i: (1, i, 0))
    row_spec = pl.BlockSpec((R, F), lambda i: (i, 0))
    full_spec = pl.BlockSpec((F, F), lambda i: (0, 0))
    bias_spec = pl.BlockSpec((1, F), lambda i: (0, 0))

    return pl.pallas_call(
        mm_body,
        grid=grid,
        in_specs=[half0_spec, half1_spec, row_spec, col0_spec, col1_spec,
                  full_spec, full_spec, bias_spec],
        out_specs=row_spec,
        out_shape=jax.ShapeDtypeStruct((N, F), jnp.float32),
    )


_tc_mm_relu = _make_tc_mm(True)
_tc_mm = _make_tc_mm(False)


def kernel(x, edge_index, W1l, W1r, b1, W2l, W2r, b2):
    src = edge_index[0].astype(jnp.int32)
    dst = edge_index[1].astype(jnp.int32)
    b1r = b1.reshape(1, F)
    b2r = b2.reshape(1, F)

    sums1, cnt = _sc_agg_counts(x, src, dst)
    cnt3 = cnt.reshape(NC, NPAD, 1)
    h = _tc_mm_relu(sums1, sums1, x, cnt3, cnt3, W1l, W1r, b1r)

    (sums2,) = _sc_agg_nocounts(h, src, dst)
    out = _tc_mm(sums2, sums2, h, cnt3, cnt3, W2l, W2r, b2r)
    return out


# speedup vs baseline: 1.1167x; 1.0396x over previous
import functools

import jax
import jax.numpy as jnp
from jax import lax
from jax.experimental import pallas as pl
from jax.experimental.pallas import tpu as pltpu
from jax.experimental.pallas import tpu_sc as plsc

N = 10000
E = 320000
F = 128

NC = 2
NS = 16
NW = NC * NS
EPT = E // NW
C = 80
NCHUNK = EPT // C
NPAD = 10240
RPT = NPAD // NS
ZROWS = 32
NBUF = 4
NIB = 5
CPT = NPAD // NS


def _make_sc_agg(with_counts: bool):
    mesh = plsc.VectorSubcoreMesh(core_axis_name="c", subcore_axis_name="s")
    out_type = [jax.ShapeDtypeStruct((NC, N, F), jnp.float32)]
    if with_counts:
        out_type.append(jax.ShapeDtypeStruct((NC, NPAD), jnp.float32))

    scratch = [
        pltpu.VMEM_SHARED((NPAD, F), jnp.float32),
        pltpu.VMEM_SHARED((NPAD,), jnp.float32),
        pltpu.VMEM((ZROWS, F), jnp.float32),
        pltpu.VMEM((CPT,), jnp.float32),
        pltpu.VMEM((NIB, C), jnp.int32),
        pltpu.VMEM((NIB, C), jnp.int32),
        pltpu.VMEM((NBUF, C, F), jnp.float32),
        pltpu.VMEM((C,), jnp.float32),
        pltpu.SemaphoreType.DMA,
        pltpu.SemaphoreType.DMA,
        pltpu.SemaphoreType.DMA,
        pltpu.SemaphoreType.DMA,
        pltpu.SemaphoreType.DMA,
    ]

    @functools.partial(
        pl.kernel, mesh=mesh, out_type=tuple(out_type),
        scratch_types=tuple(scratch),
    )
    def sc_agg(x_hbm, edge_hbm, *rest):
        if with_counts:
            sums_hbm, cnt_hbm = rest[0], rest[1]
            rest = rest[2:]
        else:
            sums_hbm, cnt_hbm = rest[0], None
            rest = rest[1:]
        (acc_sh, cnt_sh, zb, zc, sbuf, dbuf, rows, ones_v,
         isem, gsem, zsem, ssem, csem) = rest

        cid = lax.axis_index("c")
        sid = lax.axis_index("s")
        wid = sid * NC + cid

        ebase = wid * EPT

        def fire_idx(g):
            base = ebase + g * C
            b = g % NIB
            pltpu.async_copy(edge_hbm.at[pl.ds(base, C)], sbuf.at[b], isem)
            pltpu.async_copy(edge_hbm.at[pl.ds(E + base, C)], dbuf.at[b],
                             isem)

        def wait_idx():
            pltpu.make_async_copy(edge_hbm.at[pl.ds(0, C)], sbuf.at[0],
                                  isem).wait()
            pltpu.make_async_copy(edge_hbm.at[pl.ds(0, C)], dbuf.at[0],
                                  isem).wait()

        fire_idx(0)
        fire_idx(1)
        fire_idx(2)

        z16 = jnp.zeros((16,), jnp.float32)

        def zb_body(i, _):
            zb[i // (F // 16), pl.ds((i % (F // 16)) * 16, 16)] = z16
            return 0
        lax.fori_loop(0, ZROWS * (F // 16), zb_body, 0)

        if with_counts:
            def zc_body(i, _):
                zc[pl.ds(i * 16, 16)] = z16
                return 0
            lax.fori_loop(0, CPT // 16, zc_body, 0)

            def ones_body(i, _):
                ones_v[pl.ds(i * 16, 16)] = jnp.ones((16,), jnp.float32)
                return 0
            lax.fori_loop(0, C // 16, ones_body, 0)

        r0 = sid * RPT
        for r in range(RPT // ZROWS):
            pltpu.async_copy(zb, acc_sh.at[pl.ds(r0 + r * ZROWS, ZROWS)],
                             zsem)
        c0 = sid * CPT
        if with_counts:
            pltpu.async_copy(zc, cnt_sh.at[pl.ds(c0, CPT)], zsem)
        for r in range(RPT // ZROWS):
            pltpu.make_async_copy(zb, acc_sh.at[pl.ds(r0, ZROWS)],
                                  zsem).wait()
        if with_counts:
            pltpu.make_async_copy(zc, cnt_sh.at[pl.ds(c0, CPT)], zsem).wait()

        plsc.subcore_barrier()

        def fire_gather(g):
            pltpu.async_copy(x_hbm.at[sbuf.at[g % NIB]], rows.at[g % NBUF],
                             gsem)

        def wait_gather(g):
            pltpu.make_async_copy(x_hbm.at[sbuf.at[g % NIB]],
                                  rows.at[g % NBUF], gsem).wait()

        def scat(g):
            pltpu.sync_copy(rows.at[g % NBUF], acc_sh.at[dbuf.at[g % NIB]],
                            add=True)
            if with_counts:
                pltpu.sync_copy(ones_v, cnt_sh.at[dbuf.at[g % NIB]],
                                add=True)

        wait_idx()
        fire_gather(0)
        wait_idx()
        fire_gather(1)

        def chunk_body(g, _):
            wait_gather(g)

            @pl.when(g + 3 < NCHUNK)
            def _():
                fire_idx(g + 3)

            @pl.when(g + 2 < NCHUNK)
            def _():
                wait_idx()
                fire_gather(g + 2)

            scat(g)
            return 0
        lax.fori_loop(0, NCHUNK, chunk_body, 0)

        plsc.subcore_barrier()

        if with_counts:
            pltpu.sync_copy(cnt_sh.at[pl.ds(c0, CPT)],
                            cnt_hbm.at[cid, pl.ds(c0, CPT)])

        @pl.when(sid < NS - 1)
        def _():
            pltpu.sync_copy(acc_sh.at[pl.ds(r0, RPT)],
                            sums_hbm.at[cid, pl.ds(r0, RPT)])

        @pl.when(sid == NS - 1)
        def _():
            tail = N - (NS - 1) * RPT
            pltpu.sync_copy(acc_sh.at[pl.ds((NS - 1) * RPT, tail)],
                            sums_hbm.at[cid, pl.ds((NS - 1) * RPT, tail)])

    return sc_agg


_sc_agg_counts = _make_sc_agg(True)
_sc_agg_nocounts = _make_sc_agg(False)


def _make_tc_mm(relu: bool):
    R = 1000
    grid = (N // R,)

    def mm_body(s0_ref, s1_ref, x_ref, c0_ref, c1_ref, wl_ref, wr_ref, b_ref,
                o_ref):
        c = c0_ref[0] + c1_ref[0]
        scale = 1.0 / jnp.maximum(c, 1.0)
        agg = (s0_ref[0] + s1_ref[0]) * scale
        out = (jnp.dot(agg, wl_ref[...], preferred_element_type=jnp.float32)
               + jnp.dot(x_ref[...], wr_ref[...],
                         preferred_element_type=jnp.float32)
               + b_ref[...])
        if relu:
            out = jnp.maximum(out, 0.0)
        o_ref[...] = out

    half0_spec = pl.BlockSpec((1, R, F), lambda i: (0, i, 0))
    half1_spec = pl.BlockSpec((1, R, F), lambda i: (1, i, 0))
    col0_spec = pl.BlockSpec((1, R, 1), lambda i: (0, i, 0))
    col1_spec = pl.BlockSpec((1, R, 1), lambda i: (1, i, 0))
    row_spec = pl.BlockSpec((R, F), lambda i: (i, 0))
    full_spec = pl.BlockSpec((F, F), lambda i: (0, 0))
    bias_spec = pl.BlockSpec((1, F), lambda i: (0, 0))

    return pl.pallas_call(
        mm_body,
        grid=grid,
        in_specs=[half0_spec, half1_spec, row_spec, col0_spec, col1_spec,
                  full_spec, full_spec, bias_spec],
        out_specs=row_spec,
        out_shape=jax.ShapeDtypeStruct((N, F), jnp.float32),
    )


_tc_mm_relu = _make_tc_mm(True)
_tc_mm = _make_tc_mm(False)


def kernel(x, edge_index, W1l, W1r, b1, W2l, W2r, b2):
    edge_flat = edge_index.astype(jnp.int32).reshape(2 * E)
    b1r = b1.reshape(1, F)
    b2r = b2.reshape(1, F)

    sums1, cnt = _sc_agg_counts(x, edge_flat)
    cnt3 = cnt.reshape(NC, NPAD, 1)
    h = _tc_mm_relu(sums1, sums1, x, cnt3, cnt3, W1l, W1r, b1r)

    (sums2,) = _sc_agg_nocounts(h, edge_flat)
    out = _tc_mm(sums2, sums2, h, cnt3, cnt3, W2l, W2r, b2r)
    return out


# speedup vs baseline: 1.1172x; 1.0005x over previous
import functools

import jax
import jax.numpy as jnp
from jax import lax
from jax.experimental import pallas as pl
from jax.experimental.pallas import tpu as pltpu
from jax.experimental.pallas import tpu_sc as plsc

N = 10000
E = 320000
F = 128

NC = 2
NS = 16
NW = NC * NS
EPT = E // NW
C = 80
NCHUNK = EPT // C
NPAD = 10240
RPT = NPAD // NS
ZROWS = 32
NBUF = 4
NIB = 5
CPT = NPAD // NS


def _make_sc_agg(with_counts: bool):
    mesh = plsc.VectorSubcoreMesh(core_axis_name="c", subcore_axis_name="s")
    out_type = [jax.ShapeDtypeStruct((NC, N, F), jnp.float32)]
    if with_counts:
        out_type.append(jax.ShapeDtypeStruct((NC, NPAD), jnp.float32))

    scratch = [
        pltpu.VMEM_SHARED((NPAD, F), jnp.float32),
        pltpu.VMEM_SHARED((NPAD,), jnp.float32),
        pltpu.VMEM((ZROWS, F), jnp.float32),
        pltpu.VMEM((CPT,), jnp.float32),
        pltpu.VMEM((NIB, C), jnp.int32),
        pltpu.VMEM((NIB, C), jnp.int32),
        pltpu.VMEM((NBUF, C, F), jnp.float32),
        pltpu.VMEM((C,), jnp.float32),
        pltpu.SemaphoreType.DMA,
        pltpu.SemaphoreType.DMA,
        pltpu.SemaphoreType.DMA,
    ]

    @functools.partial(
        pl.kernel, mesh=mesh, out_type=tuple(out_type),
        scratch_types=tuple(scratch),
    )
    def sc_agg(x_hbm, edge_hbm, *rest):
        if with_counts:
            sums_hbm, cnt_hbm = rest[0], rest[1]
            rest = rest[2:]
        else:
            sums_hbm, cnt_hbm = rest[0], None
            rest = rest[1:]
        (acc_sh, cnt_sh, zb, zc, sbuf, dbuf, rows, ones_v,
         isem, gsem, zsem) = rest

        cid = lax.axis_index("c")
        sid = lax.axis_index("s")
        wid = sid * NC + cid

        ebase = wid * EPT

        def fire_idx(g):
            base = ebase + g * C
            b = g % NIB
            pltpu.async_copy(edge_hbm.at[pl.ds(base, C)], sbuf.at[b], isem)
            pltpu.async_copy(edge_hbm.at[pl.ds(E + base, C)], dbuf.at[b],
                             isem)

        def wait_idx():
            pltpu.make_async_copy(edge_hbm.at[pl.ds(0, C)], sbuf.at[0],
                                  isem).wait()
            pltpu.make_async_copy(edge_hbm.at[pl.ds(0, C)], dbuf.at[0],
                                  isem).wait()

        fire_idx(0)
        fire_idx(1)
        fire_idx(2)

        z16 = jnp.zeros((16,), jnp.float32)

        def zb_body(i, _):
            zb[i // (F // 16), pl.ds((i % (F // 16)) * 16, 16)] = z16
            return 0
        lax.fori_loop(0, ZROWS * (F // 16), zb_body, 0)

        if with_counts:
            def zc_body(i, _):
                zc[pl.ds(i * 16, 16)] = z16
                return 0
            lax.fori_loop(0, CPT // 16, zc_body, 0)

            def ones_body(i, _):
                ones_v[pl.ds(i * 16, 16)] = jnp.ones((16,), jnp.float32)
                return 0
            lax.fori_loop(0, C // 16, ones_body, 0)

        r0 = sid * RPT
        for r in range(RPT // ZROWS):
            pltpu.async_copy(zb, acc_sh.at[pl.ds(r0 + r * ZROWS, ZROWS)],
                             zsem)
        c0 = sid * CPT
        if with_counts:
            pltpu.async_copy(zc, cnt_sh.at[pl.ds(c0, CPT)], zsem)
        for r in range(RPT // ZROWS):
            pltpu.make_async_copy(zb, acc_sh.at[pl.ds(r0, ZROWS)],
                                  zsem).wait()
        if with_counts:
            pltpu.make_async_copy(zc, cnt_sh.at[pl.ds(c0, CPT)], zsem).wait()

        plsc.subcore_barrier()

        def fire_gather(g):
            pltpu.async_copy(x_hbm.at[sbuf.at[g % NIB]], rows.at[g % NBUF],
                             gsem)

        def wait_gather(g):
            pltpu.make_async_copy(x_hbm.at[sbuf.at[g % NIB]],
                                  rows.at[g % NBUF], gsem).wait()

        def scat(g):
            pltpu.sync_copy(rows.at[g % NBUF], acc_sh.at[dbuf.at[g % NIB]],
                            add=True)
            if with_counts:
                pltpu.sync_copy(ones_v, cnt_sh.at[dbuf.at[g % NIB]],
                                add=True)

        wait_idx()
        fire_gather(0)
        wait_idx()
        fire_gather(1)

        def chunk_body(g, _):
            wait_gather(g)

            @pl.when(g + 3 < NCHUNK)
            def _():
                fire_idx(g + 3)

            @pl.when(g + 2 < NCHUNK)
            def _():
                wait_idx()
                fire_gather(g + 2)

            scat(g)
            return 0
        lax.fori_loop(0, NCHUNK, chunk_body, 0)

        plsc.subcore_barrier()

        if with_counts:
            pltpu.sync_copy(cnt_sh.at[pl.ds(c0, CPT)],
                            cnt_hbm.at[cid, pl.ds(c0, CPT)])

        @pl.when(sid < NS - 1)
        def _():
            pltpu.sync_copy(acc_sh.at[pl.ds(r0, RPT)],
                            sums_hbm.at[cid, pl.ds(r0, RPT)])

        @pl.when(sid == NS - 1)
        def _():
            tail = N - (NS - 1) * RPT
            pltpu.sync_copy(acc_sh.at[pl.ds((NS - 1) * RPT, tail)],
                            sums_hbm.at[cid, pl.ds((NS - 1) * RPT, tail)])

    return sc_agg


_sc_agg_counts = _make_sc_agg(True)
_sc_agg_nocounts = _make_sc_agg(False)


def _make_tc_mm(relu: bool):
    R = 1000
    grid = (N // R,)

    def mm_body(s0_ref, s1_ref, x_ref, c0_ref, c1_ref, wl_ref, wr_ref, b_ref,
                o_ref):
        c = c0_ref[0] + c1_ref[0]
        scale = 1.0 / jnp.maximum(c, 1.0)
        agg = (s0_ref[0] + s1_ref[0]) * scale
        out = (jnp.dot(agg, wl_ref[...], preferred_element_type=jnp.float32)
               + jnp.dot(x_ref[...], wr_ref[...],
                         preferred_element_type=jnp.float32)
               + b_ref[...])
        if relu:
            out = jnp.maximum(out, 0.0)
        o_ref[...] = out

    half0_spec = pl.BlockSpec((1, R, F), lambda i: (0, i, 0))
    half1_spec = pl.BlockSpec((1, R, F), lambda i: (1, i, 0))
    col0_spec = pl.BlockSpec((1, R, 1), lambda i: (0, i, 0))
    col1_spec = pl.BlockSpec((1, R, 1), lambda i: (1, i, 0))
    row_spec = pl.BlockSpec((R, F), lambda i: (i, 0))
    full_spec = pl.BlockSpec((F, F), lambda i: (0, 0))
    bias_spec = pl.BlockSpec((1, F), lambda i: (0, 0))

    return pl.pallas_call(
        mm_body,
        grid=grid,
        in_specs=[half0_spec, half1_spec, row_spec, col0_spec, col1_spec,
                  full_spec, full_spec, bias_spec],
        out_specs=row_spec,
        out_shape=jax.ShapeDtypeStruct((N, F), jnp.float32),
    )


_tc_mm_relu = _make_tc_mm(True)
_tc_mm = _make_tc_mm(False)


def kernel(x, edge_index, W1l, W1r, b1, W2l, W2r, b2):
    edge_flat = edge_index.astype(jnp.int32).reshape(2 * E)
    b1r = b1.reshape(1, F)
    b2r = b2.reshape(1, F)

    sums1, cnt = _sc_agg_counts(x, edge_flat)
    cnt3 = cnt.reshape(NC, NPAD, 1)
    h = _tc_mm_relu(sums1, sums1, x, cnt3, cnt3, W1l, W1r, b1r)

    (sums2,) = _sc_agg_nocounts(h, edge_flat)
    out = _tc_mm(sums2, sums2, h, cnt3, cnt3, W2l, W2r, b2r)
    return out
